# Initial kernel scaffold; baseline (speedup 1.0000x reference)
import jax
import jax.numpy as jnp
from jax.experimental import pallas as pl


def kernel(x, edge_index, Wl1, Wr1, b1, Wl2, Wr2, b2):
    raise NotImplementedError("write your pallas kernel here")


# speedup vs baseline: 4.8163x; 4.8163x over previous
import jax
import jax.numpy as jnp
from jax import lax
from jax.experimental import pallas as pl
from jax.experimental.pallas import tpu as pltpu
from jax.experimental.pallas import tpu_sc as plsc

NC = 2
NS = 16
C = 80


def _make_sc_agg(n, e, d):
    nw = NC * NS
    ew = e // nw
    iters = ew // C
    rows_per_sub = n // NS
    zch = 80
    nz = rows_per_sub // zch

    mesh = plsc.VectorSubcoreMesh(core_axis_name="c", subcore_axis_name="s",
                                  num_cores=NC, num_subcores=NS)

    def body(x_hbm, src_hbm, dst_hbm, agg_out, src_v, dst_v, rows_v, zbuf,
             acc, sem):
        cid = lax.axis_index("c")
        sid = lax.axis_index("s")
        wid = cid * NS + sid

        z16 = jnp.zeros((16,), jnp.float32)

        def fill_z(r, _):
            def fz(i, _):
                zbuf[r, pl.ds(i * 16, 16)] = z16
                return 0
            lax.fori_loop(0, d // 16, fz, 0)
            return 0
        lax.fori_loop(0, zch, fill_z, 0)

        row0 = sid * rows_per_sub

        def zero_acc(k, _):
            pltpu.sync_copy(zbuf, acc.at[pl.ds(row0 + k * zch, zch)])
            return 0
        lax.fori_loop(0, nz, zero_acc, 0)
        plsc.subcore_barrier()

        ebase = wid * ew

        def step(j, _):
            base = ebase + j * C
            pltpu.sync_copy(src_hbm.at[pl.ds(base, C)], src_v)
            pltpu.sync_copy(dst_hbm.at[pl.ds(base, C)], dst_v)
            pltpu.async_copy(x_hbm.at[src_v], rows_v, sem).wait()
            pltpu.sync_copy(rows_v, acc.at[dst_v], add=True)
            return 0
        lax.fori_loop(0, iters, step, 0)
        plsc.subcore_barrier()

        pltpu.sync_copy(acc.at[pl.ds(row0, rows_per_sub)],
                        agg_out.at[cid, pl.ds(row0, rows_per_sub)])

    return pl.kernel(
        body,
        out_type=jax.ShapeDtypeStruct((NC, n, d), jnp.float32),
        mesh=mesh,
        scratch_types=[
            pltpu.VMEM((C,), jnp.int32),
            pltpu.VMEM((C,), jnp.int32),
            pltpu.VMEM((C, d), jnp.float32),
            pltpu.VMEM((zch, d), jnp.float32),
            pltpu.VMEM_SHARED((n, d), jnp.float32),
            pltpu.SemaphoreType.DMA,
        ],
    )


def _make_sc_counts(n, e, d):
    nw = NC * NS
    ew = e // nw
    iters = ew // C
    rows_per_sub = n // NS
    zch = 80
    nz = rows_per_sub // zch

    mesh = plsc.VectorSubcoreMesh(core_axis_name="c", subcore_axis_name="s",
                                  num_cores=NC, num_subcores=NS)

    def body(dst_hbm, cnt_out, dst_v, ones_v, zcnt, cacc):
        cid = lax.axis_index("c")
        sid = lax.axis_index("s")
        wid = cid * NS + sid

        z16 = jnp.zeros((16,), jnp.float32)
        o16 = jnp.ones((16,), jnp.float32)

        def fill(r, _):
            def fz(i, _):
                zcnt[r, pl.ds(i * 16, 16)] = z16
                return 0
            lax.fori_loop(0, d // 16, fz, 0)
            return 0
        lax.fori_loop(0, zch, fill, 0)

        def fillo(r, _):
            def fo(i, _):
                ones_v[r, pl.ds(i * 16, 16)] = o16
                return 0
            lax.fori_loop(0, d // 16, fo, 0)
            return 0
        lax.fori_loop(0, C, fillo, 0)

        row0 = sid * rows_per_sub

        def zero_acc(k, _):
            pltpu.sync_copy(zcnt, cacc.at[pl.ds(row0 + k * zch, zch)])
            return 0
        lax.fori_loop(0, nz, zero_acc, 0)
        plsc.subcore_barrier()

        ebase = wid * ew

        def step(j, _):
            base = ebase + j * C
            pltpu.sync_copy(dst_hbm.at[pl.ds(base, C)], dst_v)
            pltpu.sync_copy(ones_v, cacc.at[dst_v], add=True)
            return 0
        lax.fori_loop(0, iters, step, 0)
        plsc.subcore_barrier()

        pltpu.sync_copy(cacc.at[pl.ds(row0, rows_per_sub)],
                        cnt_out.at[cid, pl.ds(row0, rows_per_sub)])

    return pl.kernel(
        body,
        out_type=jax.ShapeDtypeStruct((NC, n, d), jnp.float32),
        mesh=mesh,
        scratch_types=[
            pltpu.VMEM((C,), jnp.int32),
            pltpu.VMEM((C, d), jnp.float32),
            pltpu.VMEM((zch, d), jnp.float32),
            pltpu.VMEM_SHARED((n, d), jnp.float32),
        ],
    )


def _dense(aggp, cntp, xin, wl, wr, b):
    n, d = xin.shape
    h = wl.shape[1]
    blk = min(1024, n)
    grid = (n // blk,)

    def body(aggp_ref, cnt_ref, x_ref, wl_ref, wr_ref, b_ref, o_ref):
        agg = aggp_ref[0] + aggp_ref[1]
        cnt = cnt_ref[0] + cnt_ref[1]
        mean = agg / jnp.maximum(cnt[:, :1], 1.0)
        acc = jnp.dot(mean, wl_ref[...], preferred_element_type=jnp.float32)
        acc = acc + jnp.dot(x_ref[...], wr_ref[...],
                            preferred_element_type=jnp.float32)
        acc = acc + b_ref[...]
        o_ref[...] = jnp.maximum(acc, 0.0)

    return pl.pallas_call(
        body,
        grid=grid,
        in_specs=[
            pl.BlockSpec((NC, blk, h), lambda i: (0, i, 0)),
            pl.BlockSpec((NC, blk, h), lambda i: (0, i, 0)),
            pl.BlockSpec((blk, d), lambda i: (i, 0)),
            pl.BlockSpec((d, h), lambda i: (0, 0)),
            pl.BlockSpec((d, h), lambda i: (0, 0)),
            pl.BlockSpec((1, h), lambda i: (0, 0)),
        ],
        out_specs=pl.BlockSpec((blk, h), lambda i: (i, 0)),
        out_shape=jax.ShapeDtypeStruct((n, h), jnp.float32),
    )(aggp, cntp, xin, wl, wr, b.reshape(1, h))


def kernel(x, edge_index, Wl1, Wr1, b1, Wl2, Wr2, b2):
    n, d = x.shape
    e = edge_index.shape[1]
    src = edge_index[0]
    dst = edge_index[1]

    npad = ((n + NS * 8 - 1) // (NS * 8)) * (NS * 8)
    npad = max(npad, ((n + 1023) // 1024) * 1024)
    x_p = jnp.pad(x, ((0, npad - n), (0, 0)))

    cnt = _make_sc_counts(npad, e, d)(dst)
    agg1 = _make_sc_agg(npad, e, d)(x_p, src, dst)
    h1 = _dense(agg1, cnt, x_p, Wl1, Wr1, b1)
    agg2 = _make_sc_agg(npad, e, Wl1.shape[1])(h1, src, dst)
    out = _dense(agg2, cnt, h1, Wl2, Wr2, b2)
    return out[:n]


# speedup vs baseline: 10.3917x; 2.1576x over previous
import jax
import jax.numpy as jnp
from jax import lax
from jax.experimental import pallas as pl
from jax.experimental.pallas import tpu as pltpu
from jax.experimental.pallas import tpu_sc as plsc

NC = 2
NS = 16
C = 80


def _fill(ref, rows, d, val):
    v16 = jnp.full((16,), val, jnp.float32)

    def fr(r, _):
        def fc(i, _):
            ref[r, pl.ds(i * 16, 16)] = v16
            return 0
        lax.fori_loop(0, d // 16, fc, 0)
        return 0
    lax.fori_loop(0, rows, fr, 0)


def _make_sc_agg(n, e, d):
    nw = NC * NS
    ew = e // nw
    iters = ew // C
    rows_per_sub = n // NS
    zch = 80
    nz = rows_per_sub // zch
    half = (iters - 1) // 2

    mesh = plsc.VectorSubcoreMesh(core_axis_name="c", subcore_axis_name="s",
                                  num_cores=NC, num_subcores=NS)

    def body(x_hbm, src_hbm, dst_hbm, agg_out, sidx_v, didx_v, rows_a,
             rows_b, acc, sem_a, sem_b):
        cid = lax.axis_index("c")
        sid = lax.axis_index("s")
        wid = cid * NS + sid

        pltpu.sync_copy(src_hbm.at[wid], sidx_v)
        pltpu.sync_copy(dst_hbm.at[wid], didx_v)

        _fill(rows_a, zch, d, 0.0)
        row0 = sid * rows_per_sub

        def zero_acc(k, _):
            pltpu.sync_copy(rows_a, acc.at[pl.ds(row0 + k * zch, zch)])
            return 0
        lax.fori_loop(0, nz, zero_acc, 0)
        plsc.subcore_barrier()

        dummy = x_hbm.at[pl.ds(0, C)]

        def gidx(j):
            return sidx_v.at[pl.ds(j * C, C)]

        pltpu.async_copy(x_hbm.at[gidx(0)], rows_a, sem_a)

        def step(k, _):
            j0 = 2 * k
            pltpu.async_copy(x_hbm.at[gidx(j0 + 1)], rows_b, sem_b)
            pltpu.make_async_copy(dummy, rows_a, sem_a).wait()
            pltpu.sync_copy(rows_a, acc.at[didx_v.at[j0]], add=True)
            pltpu.async_copy(x_hbm.at[gidx(j0 + 2)], rows_a, sem_a)
            pltpu.make_async_copy(dummy, rows_b, sem_b).wait()
            pltpu.sync_copy(rows_b, acc.at[didx_v.at[j0 + 1]], add=True)
            return 0
        lax.fori_loop(0, half, step, 0)

        pltpu.make_async_copy(dummy, rows_a, sem_a).wait()
        pltpu.sync_copy(rows_a, acc.at[didx_v.at[iters - 1]], add=True)
        plsc.subcore_barrier()

        pltpu.sync_copy(acc.at[pl.ds(row0, rows_per_sub)],
                        agg_out.at[cid, pl.ds(row0, rows_per_sub)])

    return pl.kernel(
        body,
        out_type=jax.ShapeDtypeStruct((NC, n, d), jnp.float32),
        mesh=mesh,
        scratch_types=[
            pltpu.VMEM((ew,), jnp.int32),
            pltpu.VMEM((iters, C), jnp.int32),
            pltpu.VMEM((C, d), jnp.float32),
            pltpu.VMEM((C, d), jnp.float32),
            pltpu.VMEM_SHARED((n, d), jnp.float32),
            pltpu.SemaphoreType.DMA,
            pltpu.SemaphoreType.DMA,
        ],
    )


def _make_sc_counts(n, e, d):
    nw = NC * NS
    ew = e // nw
    iters = ew // C
    rows_per_sub = n // NS
    zch = 80
    nz = rows_per_sub // zch
    burst = 5
    nb = iters // burst

    mesh = plsc.VectorSubcoreMesh(core_axis_name="c", subcore_axis_name="s",
                                  num_cores=NC, num_subcores=NS)

    def body(dst_hbm, cnt_out, didx_v, ones_v, cacc, sem_s):
        cid = lax.axis_index("c")
        sid = lax.axis_index("s")
        wid = cid * NS + sid

        pltpu.sync_copy(dst_hbm.at[wid], didx_v)

        _fill(ones_v, C, d, 0.0)
        row0 = sid * rows_per_sub

        def zero_acc(k, _):
            pltpu.sync_copy(ones_v, cacc.at[pl.ds(row0 + k * zch, zch)])
            return 0
        lax.fori_loop(0, nz, zero_acc, 0)
        _fill(ones_v, C, d, 1.0)
        plsc.subcore_barrier()

        def step(k, _):
            descs = [
                pltpu.async_copy(ones_v, cacc.at[didx_v.at[burst * k + t]],
                                 sem_s, add=True)
                for t in range(burst)
            ]
            for dsc in descs:
                dsc.wait()
            return 0
        lax.fori_loop(0, nb, step, 0)
        plsc.subcore_barrier()

        pltpu.sync_copy(cacc.at[pl.ds(row0, rows_per_sub)],
                        cnt_out.at[cid, pl.ds(row0, rows_per_sub)])

    return pl.kernel(
        body,
        out_type=jax.ShapeDtypeStruct((NC, n, d), jnp.float32),
        mesh=mesh,
        scratch_types=[
            pltpu.VMEM((iters, C), jnp.int32),
            pltpu.VMEM((C, d), jnp.float32),
            pltpu.VMEM_SHARED((n, d), jnp.float32),
            pltpu.SemaphoreType.DMA,
        ],
    )


def _dense(aggp, cntp, xin, wl, wr, b):
    n, d = xin.shape
    h = wl.shape[1]
    blk = min(1024, n)
    grid = (n // blk,)

    def body(aggp_ref, cnt_ref, x_ref, wl_ref, wr_ref, b_ref, o_ref):
        agg = aggp_ref[0] + aggp_ref[1]
        cnt = cnt_ref[0] + cnt_ref[1]
        mean = agg / jnp.maximum(cnt[:, :1], 1.0)
        acc = jnp.dot(mean, wl_ref[...], preferred_element_type=jnp.float32)
        acc = acc + jnp.dot(x_ref[...], wr_ref[...],
                            preferred_element_type=jnp.float32)
        acc = acc + b_ref[...]
        o_ref[...] = jnp.maximum(acc, 0.0)

    return pl.pallas_call(
        body,
        grid=grid,
        in_specs=[
            pl.BlockSpec((NC, blk, h), lambda i: (0, i, 0)),
            pl.BlockSpec((NC, blk, h), lambda i: (0, i, 0)),
            pl.BlockSpec((blk, d), lambda i: (i, 0)),
            pl.BlockSpec((d, h), lambda i: (0, 0)),
            pl.BlockSpec((d, h), lambda i: (0, 0)),
            pl.BlockSpec((1, h), lambda i: (0, 0)),
        ],
        out_specs=pl.BlockSpec((blk, h), lambda i: (i, 0)),
        out_shape=jax.ShapeDtypeStruct((n, h), jnp.float32),
    )(aggp, cntp, xin, wl, wr, b.reshape(1, h))


def kernel(x, edge_index, Wl1, Wr1, b1, Wl2, Wr2, b2):
    n, d = x.shape
    e = edge_index.shape[1]
    nw = NC * NS
    ew = e // nw
    iters = ew // C

    src3 = edge_index[0].reshape(nw, ew)
    dst3 = edge_index[1].reshape(nw, iters, C)

    npad = ((n + NS * 8 - 1) // (NS * 8)) * (NS * 8)
    npad = max(npad, ((n + 1023) // 1024) * 1024)
    x_p = jnp.pad(x, ((0, npad - n), (0, 0)))

    cnt = _make_sc_counts(npad, e, d)(dst3)
    agg1 = _make_sc_agg(npad, e, d)(x_p, src3, dst3)
    h1 = _dense(agg1, cnt, x_p, Wl1, Wr1, b1)
    agg2 = _make_sc_agg(npad, e, Wl1.shape[1])(h1, src3, dst3)
    out = _dense(agg2, cnt, h1, Wl2, Wr2, b2)
    return out[:n]
